# baseline (speedup 1.0000x reference)
import functools

import jax
import jax.numpy as jnp
from jax import lax
from jax.experimental import pallas as pl
from jax.experimental.pallas import tpu as pltpu
from jax.experimental.pallas import tpu_sc as plsc

N = 10000
E = 320000
D = 128
EA = 16
EPS = 1e-5

f32 = jnp.float32
bf16 = jnp.bfloat16

_NC = 2
_NS = 16
_NW = _NC * _NS

EH = E // 2

_GEW = EH // _NS
_GC = 80
_GNCH = _GEW // _GC

_SEW = EH // _NW
_SC = 40
_SNCH = _SEW // _SC

_BE = 4000
_GE = EH // _BE
_BN = 2000
_GN = N // _BN


def _softplus(x):
    return jnp.maximum(x, 0.0) + jnp.log1p(jnp.exp(-jnp.abs(x)))


def _node_precompute(x, W1a, W1c, b1):
    def body(x_ref, wa_ref, wc_ref, b1_ref, xa_ref, xc_ref):
        xv = x_ref[...]
        xa_ref[...] = jnp.dot(xv, wa_ref[...], preferred_element_type=f32) + b1_ref[...]
        xc_ref[...] = jnp.dot(xv, wc_ref[...], preferred_element_type=f32)

    return pl.pallas_call(
        body,
        grid=(_GN,),
        in_specs=[pl.BlockSpec((_BN, D), lambda i: (i, 0)),
                  pl.BlockSpec((D, D), lambda i: (0, 0)),
                  pl.BlockSpec((D, D), lambda i: (0, 0)),
                  pl.BlockSpec((1, D), lambda i: (0, 0))],
        out_specs=[pl.BlockSpec((_BN, D), lambda i: (i, 0)),
                   pl.BlockSpec((_BN, D), lambda i: (i, 0))],
        out_shape=[jax.ShapeDtypeStruct((N, D), f32),
                   jax.ShapeDtypeStruct((N, D), f32)],
    )(x, W1a, W1c, b1.reshape(1, D))


def _ea_transpose(ea_t):
    BLK = 16000

    def body(t_ref, o_ref):
        o_ref[...] = t_ref[...].T

    return pl.pallas_call(
        body,
        grid=(E // BLK,),
        in_specs=[pl.BlockSpec((EA, BLK), lambda i: (0, i))],
        out_specs=pl.BlockSpec((BLK, EA), lambda i: (i, 0)),
        out_shape=jax.ShapeDtypeStruct((E, EA), f32),
    )(ea_t)


def _edge_stage1(gpair, ea, W1b, half):
    base = half * (EH // _BE)

    def body(g_ref, ea_ref, w_ref, z1_ref, s1_ref, s2_ref):
        i = pl.program_id(0)
        z = (g_ref[0] + g_ref[1]
             + jnp.dot(ea_ref[...], w_ref[...], preferred_element_type=f32))
        z1_ref[...] = z.astype(bf16)
        zr = z.reshape(_BE // 8, 8, D)

        @pl.when(i == 0)
        def _():
            s1_ref[...] = jnp.zeros_like(s1_ref)
            s2_ref[...] = jnp.zeros_like(s2_ref)

        s1_ref[...] += jnp.sum(zr, axis=0)
        s2_ref[...] += lax.dot_general(z, z, (((0,), (0,)), ((), ())),
                                       preferred_element_type=f32)

    return pl.pallas_call(
        body,
        grid=(_GE,),
        in_specs=[pl.BlockSpec((2, _BE, D), lambda i: (0, i, 0)),
                  pl.BlockSpec((_BE, EA), lambda i: (base + i, 0)),
                  pl.BlockSpec((EA, D), lambda i: (0, 0))],
        out_specs=[pl.BlockSpec((_BE, D), lambda i: (i, 0)),
                   pl.BlockSpec((8, D), lambda i: (0, 0)),
                   pl.BlockSpec((D, D), lambda i: (0, 0))],
        out_shape=[jax.ShapeDtypeStruct((EH, D), bf16),
                   jax.ShapeDtypeStruct((8, D), f32),
                   jax.ShapeDtypeStruct((D, D), f32)],
    )(gpair, ea, W1b)


def _edge_stage2(z1, W2, b2, sc1, sh1):
    def body(z1_ref, w_ref, b2_ref, sc_ref, sh_ref, z2_ref, s1_ref, s2_ref):
        i = pl.program_id(0)
        h = _softplus(z1_ref[...].astype(f32) * sc_ref[...] + sh_ref[...])
        z2 = jnp.dot(h, w_ref[...], preferred_element_type=f32) + b2_ref[...]
        z2_ref[...] = z2.astype(bf16)
        zr = z2.reshape(_BE // 8, 8, D)

        @pl.when(i == 0)
        def _():
            s1_ref[...] = jnp.zeros_like(s1_ref)
            s2_ref[...] = jnp.zeros_like(s2_ref)

        s1_ref[...] += jnp.sum(zr, axis=0)
        s2_ref[...] += lax.dot_general(z2, z2, (((0,), (0,)), ((), ())),
                                       preferred_element_type=f32)

    return pl.pallas_call(
        body,
        grid=(_GE,),
        in_specs=[pl.BlockSpec((_BE, D), lambda i: (i, 0)),
                  pl.BlockSpec((D, D), lambda i: (0, 0)),
                  pl.BlockSpec((1, D), lambda i: (0, 0)),
                  pl.BlockSpec((1, D), lambda i: (0, 0)),
                  pl.BlockSpec((1, D), lambda i: (0, 0))],
        out_specs=[pl.BlockSpec((_BE, D), lambda i: (i, 0)),
                   pl.BlockSpec((8, D), lambda i: (0, 0)),
                   pl.BlockSpec((D, D), lambda i: (0, 0))],
        out_shape=[jax.ShapeDtypeStruct((EH, D), bf16),
                   jax.ShapeDtypeStruct((8, D), f32),
                   jax.ShapeDtypeStruct((D, D), f32)],
    )(z1, W2, b2.reshape(1, D), sc1, sh1)


def _edge_messages(z2, sc2, sh2):
    def body(z_ref, sc_ref, sh_ref, o_ref):
        o_ref[...] = _softplus(z_ref[...].astype(f32) * sc_ref[...] + sh_ref[...])

    return pl.pallas_call(
        body,
        grid=(_GE,),
        in_specs=[pl.BlockSpec((_BE, D), lambda i: (i, 0)),
                  pl.BlockSpec((1, D), lambda i: (0, 0)),
                  pl.BlockSpec((1, D), lambda i: (0, 0))],
        out_specs=pl.BlockSpec((_BE, D), lambda i: (i, 0)),
        out_shape=jax.ShapeDtypeStruct((EH, D), f32),
    )(z2, sc2, sh2)


def _node_update(x, pA, pB, Wn, bnb):
    def body(x_ref, pa_ref, pb_ref, w_ref, b_ref, z3_ref, s1_ref, s2_ref):
        i = pl.program_id(0)
        v = x_ref[...] + pa_ref[0] + pa_ref[1] + pb_ref[0] + pb_ref[1]
        z3 = jnp.dot(v, w_ref[...], preferred_element_type=f32) + b_ref[...]
        z3_ref[...] = z3
        zr = z3.reshape(_BN // 8, 8, D)

        @pl.when(i == 0)
        def _():
            s1_ref[...] = jnp.zeros_like(s1_ref)
            s2_ref[...] = jnp.zeros_like(s2_ref)

        s1_ref[...] += jnp.sum(zr, axis=0)
        s2_ref[...] += lax.dot_general(z3, z3, (((0,), (0,)), ((), ())),
                                       preferred_element_type=f32)

    return pl.pallas_call(
        body,
        grid=(_GN,),
        in_specs=[pl.BlockSpec((_BN, D), lambda i: (i, 0)),
                  pl.BlockSpec((2, _BN, D), lambda i: (0, i, 0)),
                  pl.BlockSpec((2, _BN, D), lambda i: (0, i, 0)),
                  pl.BlockSpec((D, D), lambda i: (0, 0)),
                  pl.BlockSpec((1, D), lambda i: (0, 0))],
        out_specs=[pl.BlockSpec((_BN, D), lambda i: (i, 0)),
                   pl.BlockSpec((8, D), lambda i: (0, 0)),
                   pl.BlockSpec((D, D), lambda i: (0, 0))],
        out_shape=[jax.ShapeDtypeStruct((N, D), f32),
                   jax.ShapeDtypeStruct((8, D), f32),
                   jax.ShapeDtypeStruct((D, D), f32)],
    )(x, pA, pB, Wn, bnb.reshape(1, D))


def _final_softplus(z, sc, sh):
    def body(z_ref, sc_ref, sh_ref, o_ref):
        o_ref[...] = _softplus(z_ref[...] * sc_ref[...] + sh_ref[...])

    return pl.pallas_call(
        body,
        grid=(_GN,),
        in_specs=[pl.BlockSpec((_BN, D), lambda i: (i, 0)),
                  pl.BlockSpec((1, D), lambda i: (0, 0)),
                  pl.BlockSpec((1, D), lambda i: (0, 0))],
        out_specs=pl.BlockSpec((_BN, D), lambda i: (i, 0)),
        out_shape=jax.ShapeDtypeStruct((N, D), f32),
    )(z, sc, sh)


def _gather_pairs(xa, xc, idx1, half):
    mesh = plsc.VectorSubcoreMesh(core_axis_name="c", subcore_axis_name="s")
    e0 = half * EH

    @functools.partial(
        pl.kernel,
        out_type=jax.ShapeDtypeStruct((2, EH, D), f32),
        mesh=mesh,
        scratch_types=(
            pltpu.VMEM((_GEW,), jnp.int32),
            pltpu.VMEM((_GC, D), f32),
            pltpu.VMEM((_GC, D), f32),
            pltpu.VMEM_SHARED((N, D), f32),
            pltpu.SemaphoreType.DMA,
            pltpu.SemaphoreType.DMA,
        ),
    )
    def gather_kernel(xa_hbm, xc_hbm, idx_hbm, g2_hbm, idxs, b0, b1, sp, sg, sw):
        cid = lax.axis_index("c")
        sid = lax.axis_index("s")

        @pl.when(jnp.logical_and(sid == 0, cid == 0))
        def _():
            pltpu.sync_copy(xa_hbm, sp)

        @pl.when(jnp.logical_and(sid == 0, cid == 1))
        def _():
            pltpu.sync_copy(xc_hbm, sp)

        pltpu.sync_copy(idx_hbm.at[pl.ds(cid * E + e0 + sid * _GEW, _GEW)], idxs)
        plsc.subcore_barrier()

        base = sid * _GEW
        bufs = (b0, b1)

        def do_pair(t, npair):
            descs = []
            for b in range(npair):
                j = t * 2 + b
                descs.append(pltpu.async_copy(
                    sp.at[idxs.at[pl.ds(j * _GC, _GC)]], bufs[b], sg))
            ws = []
            for b in range(npair):
                j = t * 2 + b
                off = base + j * _GC
                descs[b].wait()
                ws.append(pltpu.async_copy(bufs[b], g2_hbm.at[cid, pl.ds(off, _GC)], sw))
            for w in ws:
                w.wait()

        def body(t, carry):
            do_pair(t, 2)
            return carry

        lax.fori_loop(0, _GNCH // 2, body, 0)
        if _GNCH % 2:
            do_pair(_GNCH // 2, 1)

    return gather_kernel(xa, xc, idx1)


def _scatter_add(msg, idx1, zeros_nd, half):
    mesh = plsc.VectorSubcoreMesh(core_axis_name="c", subcore_axis_name="s")
    e0 = half * EH

    @functools.partial(
        pl.kernel,
        out_type=jax.ShapeDtypeStruct((2, N, D), f32),
        mesh=mesh,
        scratch_types=(
            pltpu.VMEM((_SEW,), jnp.int32),
            pltpu.VMEM((_SC, D), f32),
            pltpu.VMEM((_SC, D), f32),
            pltpu.VMEM_SHARED((N, D), f32),
            pltpu.SemaphoreType.DMA,
            pltpu.SemaphoreType.DMA,
        ),
    )
    def scatter_kernel(msg_hbm, idx_hbm, zeros_hbm, out_hbm,
                       cols_v, b0, b1, agg, sm, ss):
        cid = lax.axis_index("c")
        sid = lax.axis_index("s")
        wid = sid * _NC + cid

        @pl.when(sid == 0)
        def _():
            pltpu.sync_copy(zeros_hbm, agg)

        pltpu.sync_copy(idx_hbm.at[pl.ds(E + e0 + wid * _SEW, _SEW)], cols_v)
        plsc.subcore_barrier()

        base = wid * _SEW
        bufs = (b0, b1)

        def do_pair(t, npair):
            descs = []
            for b in range(npair):
                j = t * 2 + b
                off = base + j * _SC
                descs.append(pltpu.async_copy(msg_hbm.at[pl.ds(off, _SC)], bufs[b], sm))
            ws = []
            for b in range(npair):
                j = t * 2 + b
                descs[b].wait()
                ws.append(pltpu.async_copy(
                    bufs[b], agg.at[cols_v.at[pl.ds(j * _SC, _SC)]], ss, add=True))
            for w in ws:
                w.wait()

        def body(t, carry):
            do_pair(t, 2)
            return carry

        lax.fori_loop(0, _SNCH // 2, body, 0)
        if _SNCH % 2:
            do_pair(_SNCH // 2, 1)

        plsc.subcore_barrier()

        @pl.when(sid == 0)
        def _():
            pltpu.sync_copy(agg, out_hbm.at[cid])

    return scatter_kernel(msg, idx1, zeros_nd)


def _bn_coeffs(s1, gram, count, gamma, beta):
    mean = s1.sum(0) / count
    var = jnp.diagonal(gram) / count - mean * mean
    scale = gamma * lax.rsqrt(var + EPS)
    shift = beta - mean * scale
    return scale.reshape(1, D), shift.reshape(1, D)


def kernel(x, edge_index, edge_attr, W1, b1, g1, be1, W2, b2, g2, be2,
           Wn, bn, gn, ben):
    idx1 = edge_index.reshape(2 * E)
    W1a = W1[:D]
    W1b = W1[D:D + EA]
    W1c = W1[D + EA:]

    ea = _ea_transpose(edge_attr.T)
    xa, xc = _node_precompute(x, W1a, W1c, b1)
    gpA = _gather_pairs(xa, xc, idx1, 0)
    gpB = _gather_pairs(xa, xc, idx1, 1)
    z1A, s1A, s2A = _edge_stage1(gpA, ea, W1b, 0)
    z1B, s1B, s2B = _edge_stage1(gpB, ea, W1b, 1)
    sc1, sh1 = _bn_coeffs(s1A + s1B, s2A + s2B, E, g1, be1)
    z2A, t1A, t2A = _edge_stage2(z1A, W2, b2, sc1, sh1)
    z2B, t1B, t2B = _edge_stage2(z1B, W2, b2, sc1, sh1)
    sc2, sh2 = _bn_coeffs(t1A + t1B, t2A + t2B, E, g2, be2)
    zeros_nd = jnp.zeros((N, D), f32)
    msgA = _edge_messages(z2A, sc2, sh2)
    partsA = _scatter_add(msgA, idx1, zeros_nd, 0)
    msgB = _edge_messages(z2B, sc2, sh2)
    partsB = _scatter_add(msgB, idx1, zeros_nd, 1)
    z3, u1, u2 = _node_update(x, partsA, partsB, Wn, bn)
    sc3, sh3 = _bn_coeffs(u1, u2, N, gn, ben)
    return _final_softplus(z3, sc3, sh3)


# speedup vs baseline: 1.0647x; 1.0647x over previous
import functools

import jax
import jax.numpy as jnp
from jax import lax
from jax.experimental import pallas as pl
from jax.experimental.pallas import tpu as pltpu
from jax.experimental.pallas import tpu_sc as plsc

N = 10000
E = 320000
D = 128
EA = 16
EPS = 1e-5

f32 = jnp.float32
bf16 = jnp.bfloat16

_NC = 2
_NS = 16
_NW = _NC * _NS

EH = E // 2

_GEW = EH // _NS
_GC = 80
_GNCH = _GEW // _GC

_SEW = EH // _NW
_SC = 40
_SNCH = _SEW // _SC

_BE = 8000
_GE = EH // _BE
_BN = 2000
_GN = N // _BN


def _softplus(x):
    return jnp.maximum(x, 0.0) + jnp.log1p(jnp.exp(-jnp.abs(x)))


def _node_precompute(x, W1a, W1c, b1):
    def body(x_ref, wa_ref, wc_ref, b1_ref, xa_ref, xc_ref):
        xv = x_ref[...]
        xa_ref[...] = jnp.dot(xv, wa_ref[...], preferred_element_type=f32) + b1_ref[...]
        xc_ref[...] = jnp.dot(xv, wc_ref[...], preferred_element_type=f32)

    return pl.pallas_call(
        body,
        grid=(_GN,),
        in_specs=[pl.BlockSpec((_BN, D), lambda i: (i, 0)),
                  pl.BlockSpec((D, D), lambda i: (0, 0)),
                  pl.BlockSpec((D, D), lambda i: (0, 0)),
                  pl.BlockSpec((1, D), lambda i: (0, 0))],
        out_specs=[pl.BlockSpec((_BN, D), lambda i: (i, 0)),
                   pl.BlockSpec((_BN, D), lambda i: (i, 0))],
        out_shape=[jax.ShapeDtypeStruct((N, D), f32),
                   jax.ShapeDtypeStruct((N, D), f32)],
    )(x, W1a, W1c, b1.reshape(1, D))


def _ea_transpose(ea_t):
    BLK = 16000

    def body(t_ref, o_ref):
        o_ref[...] = t_ref[...].T

    return pl.pallas_call(
        body,
        grid=(E // BLK,),
        in_specs=[pl.BlockSpec((EA, BLK), lambda i: (0, i))],
        out_specs=pl.BlockSpec((BLK, EA), lambda i: (i, 0)),
        out_shape=jax.ShapeDtypeStruct((E, EA), f32),
    )(ea_t)


def _edge_stage1(gpair, ea, W1b, half):
    base = half * (EH // _BE)

    def body(g_ref, ea_ref, w_ref, z1_ref, s1_ref, s2_ref):
        i = pl.program_id(0)
        z = (g_ref[0] + g_ref[1]
             + jnp.dot(ea_ref[...], w_ref[...], preferred_element_type=f32))
        z1_ref[...] = z.astype(bf16)
        zr = z.reshape(_BE // 8, 8, D)

        @pl.when(i == 0)
        def _():
            s1_ref[...] = jnp.zeros_like(s1_ref)
            s2_ref[...] = jnp.zeros_like(s2_ref)

        s1_ref[...] += jnp.sum(zr, axis=0)
        s2_ref[...] += jnp.sum(zr * zr, axis=0)

    return pl.pallas_call(
        body,
        grid=(_GE,),
        in_specs=[pl.BlockSpec((2, _BE, D), lambda i: (0, i, 0)),
                  pl.BlockSpec((_BE, EA), lambda i: (base + i, 0)),
                  pl.BlockSpec((EA, D), lambda i: (0, 0))],
        out_specs=[pl.BlockSpec((_BE, D), lambda i: (i, 0)),
                   pl.BlockSpec((8, D), lambda i: (0, 0)),
                   pl.BlockSpec((8, D), lambda i: (0, 0))],
        out_shape=[jax.ShapeDtypeStruct((EH, D), bf16),
                   jax.ShapeDtypeStruct((8, D), f32),
                   jax.ShapeDtypeStruct((8, D), f32)],
    )(gpair, ea, W1b)


def _edge_stage2(z1, W2, b2, sc1, sh1):
    def body(z1_ref, w_ref, b2_ref, sc_ref, sh_ref, z2_ref, s1_ref, s2_ref):
        i = pl.program_id(0)
        h = _softplus(z1_ref[...].astype(f32) * sc_ref[...] + sh_ref[...])
        z2 = jnp.dot(h, w_ref[...], preferred_element_type=f32) + b2_ref[...]
        z2_ref[...] = z2.astype(bf16)
        zr = z2.reshape(_BE // 8, 8, D)

        @pl.when(i == 0)
        def _():
            s1_ref[...] = jnp.zeros_like(s1_ref)
            s2_ref[...] = jnp.zeros_like(s2_ref)

        s1_ref[...] += jnp.sum(zr, axis=0)
        s2_ref[...] += jnp.sum(zr * zr, axis=0)

    return pl.pallas_call(
        body,
        grid=(_GE,),
        in_specs=[pl.BlockSpec((_BE, D), lambda i: (i, 0)),
                  pl.BlockSpec((D, D), lambda i: (0, 0)),
                  pl.BlockSpec((1, D), lambda i: (0, 0)),
                  pl.BlockSpec((1, D), lambda i: (0, 0)),
                  pl.BlockSpec((1, D), lambda i: (0, 0))],
        out_specs=[pl.BlockSpec((_BE, D), lambda i: (i, 0)),
                   pl.BlockSpec((8, D), lambda i: (0, 0)),
                   pl.BlockSpec((8, D), lambda i: (0, 0))],
        out_shape=[jax.ShapeDtypeStruct((EH, D), bf16),
                   jax.ShapeDtypeStruct((8, D), f32),
                   jax.ShapeDtypeStruct((8, D), f32)],
    )(z1, W2, b2.reshape(1, D), sc1, sh1)


def _edge_messages(z2, sc2, sh2):
    def body(z_ref, sc_ref, sh_ref, o_ref):
        o_ref[...] = _softplus(z_ref[...].astype(f32) * sc_ref[...] + sh_ref[...])

    return pl.pallas_call(
        body,
        grid=(_GE,),
        in_specs=[pl.BlockSpec((_BE, D), lambda i: (i, 0)),
                  pl.BlockSpec((1, D), lambda i: (0, 0)),
                  pl.BlockSpec((1, D), lambda i: (0, 0))],
        out_specs=pl.BlockSpec((_BE, D), lambda i: (i, 0)),
        out_shape=jax.ShapeDtypeStruct((EH, D), f32),
    )(z2, sc2, sh2)


def _node_update(x, pA, pB, Wn, bnb):
    def body(x_ref, pa_ref, pb_ref, w_ref, b_ref, z3_ref, s1_ref, s2_ref):
        i = pl.program_id(0)
        v = x_ref[...] + pa_ref[0] + pa_ref[1] + pb_ref[0] + pb_ref[1]
        z3 = jnp.dot(v, w_ref[...], preferred_element_type=f32) + b_ref[...]
        z3_ref[...] = z3
        zr = z3.reshape(_BN // 8, 8, D)

        @pl.when(i == 0)
        def _():
            s1_ref[...] = jnp.zeros_like(s1_ref)
            s2_ref[...] = jnp.zeros_like(s2_ref)

        s1_ref[...] += jnp.sum(zr, axis=0)
        s2_ref[...] += jnp.sum(zr * zr, axis=0)

    return pl.pallas_call(
        body,
        grid=(_GN,),
        in_specs=[pl.BlockSpec((_BN, D), lambda i: (i, 0)),
                  pl.BlockSpec((2, _BN, D), lambda i: (0, i, 0)),
                  pl.BlockSpec((2, _BN, D), lambda i: (0, i, 0)),
                  pl.BlockSpec((D, D), lambda i: (0, 0)),
                  pl.BlockSpec((1, D), lambda i: (0, 0))],
        out_specs=[pl.BlockSpec((_BN, D), lambda i: (i, 0)),
                   pl.BlockSpec((8, D), lambda i: (0, 0)),
                   pl.BlockSpec((8, D), lambda i: (0, 0))],
        out_shape=[jax.ShapeDtypeStruct((N, D), f32),
                   jax.ShapeDtypeStruct((8, D), f32),
                   jax.ShapeDtypeStruct((8, D), f32)],
    )(x, pA, pB, Wn, bnb.reshape(1, D))


def _final_softplus(z, sc, sh):
    def body(z_ref, sc_ref, sh_ref, o_ref):
        o_ref[...] = _softplus(z_ref[...] * sc_ref[...] + sh_ref[...])

    return pl.pallas_call(
        body,
        grid=(_GN,),
        in_specs=[pl.BlockSpec((_BN, D), lambda i: (i, 0)),
                  pl.BlockSpec((1, D), lambda i: (0, 0)),
                  pl.BlockSpec((1, D), lambda i: (0, 0))],
        out_specs=pl.BlockSpec((_BN, D), lambda i: (i, 0)),
        out_shape=jax.ShapeDtypeStruct((N, D), f32),
    )(z, sc, sh)


def _gather_pairs(xa, xc, idx1, half):
    mesh = plsc.VectorSubcoreMesh(core_axis_name="c", subcore_axis_name="s")
    e0 = half * EH

    @functools.partial(
        pl.kernel,
        out_type=jax.ShapeDtypeStruct((2, EH, D), f32),
        mesh=mesh,
        scratch_types=(
            pltpu.VMEM((_GEW,), jnp.int32),
            pltpu.VMEM((_GC, D), f32),
            pltpu.VMEM((_GC, D), f32),
            pltpu.VMEM_SHARED((N, D), f32),
            pltpu.SemaphoreType.DMA,
            pltpu.SemaphoreType.DMA,
        ),
    )
    def gather_kernel(xa_hbm, xc_hbm, idx_hbm, g2_hbm, idxs, b0, b1, sp, sg, sw):
        cid = lax.axis_index("c")
        sid = lax.axis_index("s")

        @pl.when(jnp.logical_and(sid == 0, cid == 0))
        def _():
            pltpu.sync_copy(xa_hbm, sp)

        @pl.when(jnp.logical_and(sid == 0, cid == 1))
        def _():
            pltpu.sync_copy(xc_hbm, sp)

        pltpu.sync_copy(idx_hbm.at[pl.ds(cid * E + e0 + sid * _GEW, _GEW)], idxs)
        plsc.subcore_barrier()

        base = sid * _GEW
        bufs = (b0, b1)

        def do_pair(t, npair):
            descs = []
            for b in range(npair):
                j = t * 2 + b
                descs.append(pltpu.async_copy(
                    sp.at[idxs.at[pl.ds(j * _GC, _GC)]], bufs[b], sg))
            ws = []
            for b in range(npair):
                j = t * 2 + b
                off = base + j * _GC
                descs[b].wait()
                ws.append(pltpu.async_copy(bufs[b], g2_hbm.at[cid, pl.ds(off, _GC)], sw))
            for w in ws:
                w.wait()

        def body(t, carry):
            do_pair(t, 2)
            return carry

        lax.fori_loop(0, _GNCH // 2, body, 0)
        if _GNCH % 2:
            do_pair(_GNCH // 2, 1)

    return gather_kernel(xa, xc, idx1)


def _scatter_add(msg, idx1, zeros_nd, half):
    mesh = plsc.VectorSubcoreMesh(core_axis_name="c", subcore_axis_name="s")
    e0 = half * EH

    @functools.partial(
        pl.kernel,
        out_type=jax.ShapeDtypeStruct((2, N, D), f32),
        mesh=mesh,
        scratch_types=(
            pltpu.VMEM((_SEW,), jnp.int32),
            pltpu.VMEM((_SC, D), f32),
            pltpu.VMEM((_SC, D), f32),
            pltpu.VMEM_SHARED((N, D), f32),
            pltpu.SemaphoreType.DMA,
            pltpu.SemaphoreType.DMA,
        ),
    )
    def scatter_kernel(msg_hbm, idx_hbm, zeros_hbm, out_hbm,
                       cols_v, b0, b1, agg, sm, ss):
        cid = lax.axis_index("c")
        sid = lax.axis_index("s")
        wid = sid * _NC + cid

        @pl.when(sid == 0)
        def _():
            pltpu.sync_copy(zeros_hbm, agg)

        pltpu.sync_copy(idx_hbm.at[pl.ds(E + e0 + wid * _SEW, _SEW)], cols_v)
        plsc.subcore_barrier()

        base = wid * _SEW
        bufs = (b0, b1)

        def do_pair(t, npair):
            descs = []
            for b in range(npair):
                j = t * 2 + b
                off = base + j * _SC
                descs.append(pltpu.async_copy(msg_hbm.at[pl.ds(off, _SC)], bufs[b], sm))
            ws = []
            for b in range(npair):
                j = t * 2 + b
                descs[b].wait()
                ws.append(pltpu.async_copy(
                    bufs[b], agg.at[cols_v.at[pl.ds(j * _SC, _SC)]], ss, add=True))
            for w in ws:
                w.wait()

        def body(t, carry):
            do_pair(t, 2)
            return carry

        lax.fori_loop(0, _SNCH // 2, body, 0)
        if _SNCH % 2:
            do_pair(_SNCH // 2, 1)

        plsc.subcore_barrier()

        @pl.when(sid == 0)
        def _():
            pltpu.sync_copy(agg, out_hbm.at[cid])

    return scatter_kernel(msg, idx1, zeros_nd)


def _bn_coeffs(s1, s2, count, gamma, beta):
    mean = s1.sum(0) / count
    var = s2.sum(0) / count - mean * mean
    scale = gamma * lax.rsqrt(var + EPS)
    shift = beta - mean * scale
    return scale.reshape(1, D), shift.reshape(1, D)


def kernel(x, edge_index, edge_attr, W1, b1, g1, be1, W2, b2, g2, be2,
           Wn, bn, gn, ben):
    idx1 = edge_index.reshape(2 * E)
    W1a = W1[:D]
    W1b = W1[D:D + EA]
    W1c = W1[D + EA:]

    ea = _ea_transpose(edge_attr.T)
    xa, xc = _node_precompute(x, W1a, W1c, b1)
    gpA = _gather_pairs(xa, xc, idx1, 0)
    gpB = _gather_pairs(xa, xc, idx1, 1)
    z1A, s1A, s2A = _edge_stage1(gpA, ea, W1b, 0)
    z1B, s1B, s2B = _edge_stage1(gpB, ea, W1b, 1)
    sc1, sh1 = _bn_coeffs(s1A + s1B, s2A + s2B, E, g1, be1)
    z2A, t1A, t2A = _edge_stage2(z1A, W2, b2, sc1, sh1)
    z2B, t1B, t2B = _edge_stage2(z1B, W2, b2, sc1, sh1)
    sc2, sh2 = _bn_coeffs(t1A + t1B, t2A + t2B, E, g2, be2)
    zeros_nd = jnp.zeros((N, D), f32)
    msgA = _edge_messages(z2A, sc2, sh2)
    partsA = _scatter_add(msgA, idx1, zeros_nd, 0)
    msgB = _edge_messages(z2B, sc2, sh2)
    partsB = _scatter_add(msgB, idx1, zeros_nd, 1)
    z3, u1, u2 = _node_update(x, partsA, partsB, Wn, bn)
    sc3, sh3 = _bn_coeffs(u1, u2, N, gn, ben)
    return _final_softplus(z3, sc3, sh3)
